# Initial kernel scaffold; baseline (speedup 1.0000x reference)
import functools

import jax
import jax.numpy as jnp
from jax import lax
from jax.experimental import pallas as pl
from jax.experimental.pallas import tpu as pltpu
from jax.experimental.pallas import tpu_sc as plsc

N = 10000
D = 256
K = 8
DELTA = D // K
E = 160000

NC = 2
NS = 16
H = D // 2

ROWS_PER_TILE = N // NS

PC = 80
PCHUNKS = E // NS // PC

DC = 40
DCHUNKS = E // (NC * NS) // DC
DEG_W = 16

_MESH = plsc.VectorSubcoreMesh(core_axis_name="c", subcore_axis_name="s")


@functools.partial(
    pl.kernel,
    out_type=jax.ShapeDtypeStruct((NC * N, DEG_W), jnp.float32),
    mesh=_MESH,
    scratch_types=[
        pltpu.VMEM((DCHUNKS, DC), jnp.int32),
        pltpu.VMEM((DC, DEG_W), jnp.float32),
        pltpu.VMEM_SHARED((N, DEG_W), jnp.float32),
    ],
)
def _sc_degree(dst_hbm, ones_hbm, zeros_hbm, out_hbm, idx_v, ones_v, acc):
    c = lax.axis_index("c")
    s = lax.axis_index("s")
    wid = c * NS + s
    pltpu.sync_copy(zeros_hbm, acc.at[pl.ds(s * ROWS_PER_TILE, ROWS_PER_TILE)])
    pltpu.sync_copy(dst_hbm.at[wid], idx_v)
    pltpu.sync_copy(ones_hbm, ones_v)
    plsc.subcore_barrier()

    @pl.loop(0, DCHUNKS)
    def _(j):
        pltpu.sync_copy(ones_v, acc.at[idx_v.at[j]], add=True)

    plsc.subcore_barrier()
    base = c * N + s * ROWS_PER_TILE
    pltpu.sync_copy(acc.at[pl.ds(s * ROWS_PER_TILE, ROWS_PER_TILE)],
                    out_hbm.at[pl.ds(base, ROWS_PER_TILE)])


@functools.partial(
    pl.kernel,
    out_type=jax.ShapeDtypeStruct((NC * N, H), jnp.float32),
    mesh=_MESH,
    scratch_types=[
        pltpu.VMEM((PCHUNKS, PC), jnp.int32),
        pltpu.VMEM((PCHUNKS, PC), jnp.int32),
        pltpu.VMEM((PC, H), jnp.float32),
        pltpu.VMEM_SHARED((N, H), jnp.float32),
        pltpu.SemaphoreType.DMA,
    ],
)
def _sc_propagate(y_hbm, src_hbm, dst_hbm, zeros_hbm, out_hbm,
                  src_v, dst_v, rows_v, acc, sem):
    c = lax.axis_index("c")
    s = lax.axis_index("s")
    wid = c * NS + s
    pltpu.sync_copy(zeros_hbm, acc.at[pl.ds(s * ROWS_PER_TILE, ROWS_PER_TILE)])
    pltpu.sync_copy(src_hbm.at[wid], src_v)
    pltpu.sync_copy(dst_hbm.at[s], dst_v)
    plsc.subcore_barrier()

    @pl.loop(0, PCHUNKS)
    def _(j):
        pltpu.async_copy(y_hbm.at[src_v.at[j]], rows_v, sem).wait()
        pltpu.sync_copy(rows_v, acc.at[dst_v.at[j]], add=True)

    plsc.subcore_barrier()
    base = c * N + s * ROWS_PER_TILE
    pltpu.sync_copy(acc.at[pl.ds(s * ROWS_PER_TILE, ROWS_PER_TILE)],
                    out_hbm.at[pl.ds(base, ROWS_PER_TILE)])


BN = 256
GRID = N // BN


def _dinv_from_deg(deg_ref):
    deg = deg_ref[0, :, 0:1] + deg_ref[1, :, 0:1] + 1.0
    return lax.rsqrt(deg)


def _tc_scale_body(deg_ref, x_ref, y_ref):
    dinv = _dinv_from_deg(deg_ref)
    y = dinv * x_ref[...]
    y_ref[0] = y[:, :H]
    y_ref[1] = y[:, H:]


def _tc_mid_body(deg_ref, s1_ref, x_ref, w1_ref, b1_ref, w2_ref,
                 g_ref, y2_ref):
    dinv = _dinv_from_deg(deg_ref)
    s1 = jnp.concatenate([s1_ref[0], s1_ref[1]], axis=1)
    a = dinv * s1 + (dinv * dinv) * x_ref[...]
    h = jnp.maximum(
        jnp.dot(a, w1_ref[...], preferred_element_type=jnp.float32)
        + b1_ref[...], 0.0)
    g = jnp.dot(h, w2_ref[...], preferred_element_type=jnp.float32)
    g_ref[...] = g
    y2 = dinv * g
    y2_ref[0] = y2[:, :H]
    y2_ref[1] = y2[:, H:]


def _tc_final_body(deg_ref, s2_ref, g_ref, b2_ref, out_ref):
    dinv = _dinv_from_deg(deg_ref)
    s2 = jnp.concatenate([s2_ref[0], s2_ref[1]], axis=1)
    pre = dinv * s2 + (dinv * dinv) * g_ref[...] + b2_ref[...]
    row = lax.broadcasted_iota(jnp.int32, (D, K), 0) // DELTA
    col = lax.broadcasted_iota(jnp.int32, (D, K), 1)
    m = (row == col).astype(jnp.float32)
    ssq = jnp.dot(pre * pre, m, preferred_element_type=jnp.float32)
    rnorm = 1.0 / jnp.maximum(jnp.sqrt(ssq), 1e-12)
    scale = jnp.dot(rnorm, m.T, preferred_element_type=jnp.float32)
    out_ref[...] = pre * scale


_NBLK = N // BN if N % BN == 0 else N // BN + 1

_DEG_SPEC = pl.BlockSpec((2, BN, DEG_W), lambda i: (0, i, 0))
_CAT_SPEC = pl.BlockSpec((2, BN, H), lambda i: (0, i, 0))
_X_SPEC = pl.BlockSpec((BN, D), lambda i: (i, 0))


def _tc_scale(deg, x):
    return pl.pallas_call(
        _tc_scale_body,
        grid=(_NBLK,),
        in_specs=[_DEG_SPEC, _X_SPEC],
        out_specs=_CAT_SPEC,
        out_shape=jax.ShapeDtypeStruct((2, N, H), jnp.float32),
    )(deg, x)


def _tc_mid(deg, s1, x, w1, b1, w2):
    return pl.pallas_call(
        _tc_mid_body,
        grid=(_NBLK,),
        in_specs=[
            _DEG_SPEC, _CAT_SPEC, _X_SPEC,
            pl.BlockSpec((D, 4 * D), lambda i: (0, 0)),
            pl.BlockSpec((1, 4 * D), lambda i: (0, 0)),
            pl.BlockSpec((4 * D, D), lambda i: (0, 0)),
        ],
        out_specs=[_X_SPEC, _CAT_SPEC],
        out_shape=[
            jax.ShapeDtypeStruct((N, D), jnp.float32),
            jax.ShapeDtypeStruct((2, N, H), jnp.float32),
        ],
    )(deg, s1, x, w1, b1, w2)


def _tc_final(deg, s2, g, b2):
    return pl.pallas_call(
        _tc_final_body,
        grid=(_NBLK,),
        in_specs=[
            _DEG_SPEC, _CAT_SPEC, _X_SPEC,
            pl.BlockSpec((1, D), lambda i: (0, 0)),
        ],
        out_specs=_X_SPEC,
        out_shape=jax.ShapeDtypeStruct((N, D), jnp.float32),
    )(deg, s2, g, b2)


def kernel(x_all, max_iter, ix, edge_index, aug_loss, W1, b1, W2, b2):
    src = edge_index[0]
    dst = edge_index[1]

    src_r = src.reshape(NS, PCHUNKS, PC)
    src_arr = jnp.concatenate([src_r, src_r + N], axis=0)
    dst_arr = dst.reshape(NS, PCHUNKS, PC)
    dst_deg = dst.reshape(NC * NS, DCHUNKS, DC)

    ones_deg = jnp.ones((DC, DEG_W), jnp.float32)
    zeros_deg = jnp.zeros((ROWS_PER_TILE, DEG_W), jnp.float32)
    zeros_prop = jnp.zeros((ROWS_PER_TILE, H), jnp.float32)

    deg = _sc_degree(dst_deg, ones_deg, zeros_deg).reshape(NC, N, DEG_W)

    y1 = _tc_scale(deg, x_all).reshape(NC * N, H)
    s1 = _sc_propagate(y1, src_arr, dst_arr, zeros_prop).reshape(NC, N, H)

    g, y2 = _tc_mid(deg, s1, x_all, W1, b1.reshape(1, 4 * D), W2)
    s2 = _sc_propagate(y2.reshape(NC * N, H), src_arr, dst_arr,
                       zeros_prop).reshape(NC, N, H)

    return _tc_final(deg, s2, g, b2.reshape(1, D))


# speedup vs baseline: 13.2069x; 13.2069x over previous
import functools

import jax
import jax.numpy as jnp
from jax import lax
from jax.experimental import pallas as pl
from jax.experimental.pallas import tpu as pltpu
from jax.experimental.pallas import tpu_sc as plsc

N = 10000
D = 256
K = 8
DELTA = D // K
E = 160000

NC = 2
NS = 16
H = D // 2

NP = 10240
ROWS_PER_TILE = NP // NS

PC = 80
PCHUNKS = E // NS // PC

DC = 40
DCHUNKS = E // (NC * NS) // DC
DEG_W = 16


@functools.lru_cache(maxsize=1)
def _sc_kernels():
    mesh = plsc.VectorSubcoreMesh(core_axis_name="c", subcore_axis_name="s",
                                  num_cores=NC, num_subcores=NS)

    @functools.partial(
        pl.kernel,
        out_type=jax.ShapeDtypeStruct((NC * NP, DEG_W), jnp.float32),
        mesh=mesh,
        scratch_types=[
            pltpu.VMEM((DCHUNKS, DC), jnp.int32),
            pltpu.VMEM((DC, DEG_W), jnp.float32),
            pltpu.VMEM_SHARED((NP, DEG_W), jnp.float32),
        ],
    )
    def sc_degree(dst_hbm, ones_hbm, zeros_hbm, out_hbm, idx_v, ones_v, acc):
        c = lax.axis_index("c")
        s = lax.axis_index("s")
        wid = c * NS + s
        pltpu.sync_copy(zeros_hbm,
                        acc.at[pl.ds(s * ROWS_PER_TILE, ROWS_PER_TILE)])
        pltpu.sync_copy(dst_hbm.at[wid], idx_v)
        pltpu.sync_copy(ones_hbm, ones_v)
        plsc.subcore_barrier()

        @pl.loop(0, DCHUNKS)
        def _(j):
            pltpu.sync_copy(ones_v, acc.at[idx_v.at[j]], add=True)

        plsc.subcore_barrier()
        base = c * NP + s * ROWS_PER_TILE
        pltpu.sync_copy(acc.at[pl.ds(s * ROWS_PER_TILE, ROWS_PER_TILE)],
                        out_hbm.at[pl.ds(base, ROWS_PER_TILE)])

    @functools.partial(
        pl.kernel,
        out_type=jax.ShapeDtypeStruct((NC * NP, H), jnp.float32),
        mesh=mesh,
        scratch_types=[
            pltpu.VMEM((PCHUNKS, PC), jnp.int32),
            pltpu.VMEM((PCHUNKS, PC), jnp.int32),
            pltpu.VMEM((PC, H), jnp.float32),
            pltpu.VMEM_SHARED((NP, H), jnp.float32),
            pltpu.SemaphoreType.DMA,
        ],
    )
    def sc_propagate(y_hbm, src_hbm, dst_hbm, zeros_hbm, out_hbm,
                     src_v, dst_v, rows_v, acc, sem):
        c = lax.axis_index("c")
        s = lax.axis_index("s")
        wid = c * NS + s
        pltpu.sync_copy(zeros_hbm,
                        acc.at[pl.ds(s * ROWS_PER_TILE, ROWS_PER_TILE)])
        pltpu.sync_copy(src_hbm.at[wid], src_v)
        pltpu.sync_copy(dst_hbm.at[s], dst_v)
        plsc.subcore_barrier()

        @pl.loop(0, PCHUNKS)
        def _(j):
            pltpu.async_copy(y_hbm.at[src_v.at[j]], rows_v, sem).wait()
            pltpu.sync_copy(rows_v, acc.at[dst_v.at[j]], add=True)

        plsc.subcore_barrier()
        base = c * NP + s * ROWS_PER_TILE
        pltpu.sync_copy(acc.at[pl.ds(s * ROWS_PER_TILE, ROWS_PER_TILE)],
                        out_hbm.at[pl.ds(base, ROWS_PER_TILE)])

    return sc_degree, sc_propagate


BN = 400


def _dinv_from_deg(deg_ref):
    deg = deg_ref[0, :, 0:1] + deg_ref[1, :, 0:1] + 1.0
    return lax.rsqrt(deg)


def _tc_scale_body(deg_ref, x_ref, y_ref):
    dinv = _dinv_from_deg(deg_ref)
    y = dinv * x_ref[...]
    y_ref[0] = y[:, :H]
    y_ref[1] = y[:, H:]


def _tc_mid_body(deg_ref, s1_ref, x_ref, w1_ref, b1_ref, w2_ref,
                 g_ref, y2_ref):
    dinv = _dinv_from_deg(deg_ref)
    s1 = jnp.concatenate([s1_ref[0], s1_ref[1]], axis=1)
    a = dinv * s1 + (dinv * dinv) * x_ref[...]
    h = jnp.maximum(
        jnp.dot(a, w1_ref[...], preferred_element_type=jnp.float32)
        + b1_ref[...], 0.0)
    g = jnp.dot(h, w2_ref[...], preferred_element_type=jnp.float32)
    g_ref[...] = g
    y2 = dinv * g
    y2_ref[0] = y2[:, :H]
    y2_ref[1] = y2[:, H:]


def _tc_final_body(deg_ref, s2_ref, g_ref, b2_ref, out_ref):
    dinv = _dinv_from_deg(deg_ref)
    s2 = jnp.concatenate([s2_ref[0], s2_ref[1]], axis=1)
    pre = dinv * s2 + (dinv * dinv) * g_ref[...] + b2_ref[...]
    row = lax.broadcasted_iota(jnp.int32, (D, K), 0) // DELTA
    col = lax.broadcasted_iota(jnp.int32, (D, K), 1)
    m = (row == col).astype(jnp.float32)
    ssq = jnp.dot(pre * pre, m, preferred_element_type=jnp.float32)
    rnorm = 1.0 / jnp.maximum(jnp.sqrt(ssq), 1e-12)
    scale = jnp.dot(rnorm, m.T, preferred_element_type=jnp.float32)
    out_ref[...] = pre * scale


_NBLK = N // BN

_DEG_SPEC = pl.BlockSpec((2, BN, DEG_W), lambda i: (0, i, 0))
_CAT_SPEC = pl.BlockSpec((2, BN, H), lambda i: (0, i, 0))
_X_SPEC = pl.BlockSpec((BN, D), lambda i: (i, 0))


def _tc_scale(deg, x):
    return pl.pallas_call(
        _tc_scale_body,
        grid=(_NBLK,),
        in_specs=[_DEG_SPEC, _X_SPEC],
        out_specs=_CAT_SPEC,
        out_shape=jax.ShapeDtypeStruct((2, NP, H), jnp.float32),
    )(deg, x)


def _tc_mid(deg, s1, x, w1, b1, w2):
    return pl.pallas_call(
        _tc_mid_body,
        grid=(_NBLK,),
        in_specs=[
            _DEG_SPEC, _CAT_SPEC, _X_SPEC,
            pl.BlockSpec((D, 4 * D), lambda i: (0, 0)),
            pl.BlockSpec((1, 4 * D), lambda i: (0, 0)),
            pl.BlockSpec((4 * D, D), lambda i: (0, 0)),
        ],
        out_specs=[_X_SPEC, _CAT_SPEC],
        out_shape=[
            jax.ShapeDtypeStruct((N, D), jnp.float32),
            jax.ShapeDtypeStruct((2, NP, H), jnp.float32),
        ],
    )(deg, s1, x, w1, b1, w2)


def _tc_final(deg, s2, g, b2):
    return pl.pallas_call(
        _tc_final_body,
        grid=(_NBLK,),
        in_specs=[
            _DEG_SPEC, _CAT_SPEC, _X_SPEC,
            pl.BlockSpec((1, D), lambda i: (0, 0)),
        ],
        out_specs=_X_SPEC,
        out_shape=jax.ShapeDtypeStruct((N, D), jnp.float32),
    )(deg, s2, g, b2)


def kernel(x_all, max_iter, ix, edge_index, aug_loss, W1, b1, W2, b2):
    src = edge_index[0]
    dst = edge_index[1]

    src_r = src.reshape(NS, PCHUNKS, PC)
    src_arr = jnp.concatenate([src_r, src_r + NP], axis=0)
    dst_arr = dst.reshape(NS, PCHUNKS, PC)
    dst_deg = dst.reshape(NC * NS, DCHUNKS, DC)

    ones_deg = jnp.ones((DC, DEG_W), jnp.float32)
    zeros_deg = jnp.zeros((ROWS_PER_TILE, DEG_W), jnp.float32)
    zeros_prop = jnp.zeros((ROWS_PER_TILE, H), jnp.float32)

    sc_degree, sc_propagate = _sc_kernels()
    deg = sc_degree(dst_deg, ones_deg, zeros_deg).reshape(NC, NP, DEG_W)

    y1 = _tc_scale(deg, x_all).reshape(NC * NP, H)
    s1 = sc_propagate(y1, src_arr, dst_arr, zeros_prop).reshape(NC, NP, H)

    g, y2 = _tc_mid(deg, s1, x_all, W1, b1.reshape(1, 4 * D), W2)
    s2 = sc_propagate(y2.reshape(NC * NP, H), src_arr, dst_arr,
                      zeros_prop).reshape(NC, NP, H)

    return _tc_final(deg, s2, g, b2.reshape(1, D))
